# baseline (speedup 1.0000x reference)
import functools

import jax
import jax.numpy as jnp
from jax import lax
from jax.experimental import pallas as pl
from jax.experimental.pallas import tpu as pltpu
from jax.experimental.pallas import tpu_sc as plsc

NUM_VARS = 512
HIDDEN = 16
TOPK = 8
TEMP = 1.0

_NC = 2
_NS = 16
_LANES = 16
_ROWS_PER_W = NUM_VARS // (_NC * _NS)


def _sim_kernel(ve_ref, wq_ref, bq_ref, wk_ref, bk_ref, sim_ref):
    ve = ve_ref[...]
    q = lax.dot_general(ve, wq_ref[...], (((1,), (1,)), ((), ())),
                        preferred_element_type=jnp.float32) + bq_ref[...]
    k = lax.dot_general(ve, wk_ref[...], (((1,), (1,)), ((), ())),
                        preferred_element_type=jnp.float32) + bk_ref[...]
    sim = lax.dot_general(q, k, (((1,), (1,)), ((), ())),
                          preferred_element_type=jnp.float32)
    n = sim.shape[0]
    row = lax.broadcasted_iota(jnp.int32, (n, n), 0)
    col = lax.broadcasted_iota(jnp.int32, (n, n), 1)
    sim_ref[...] = jnp.where(row == col, jnp.float32(-1e9), sim)


def _sc_routing_body(sim_hbm, s_hbm, sim_tile, s_tile):
    wid = lax.axis_index("s") * _NC + lax.axis_index("c")
    base = wid * _ROWS_PER_W
    pltpu.sync_copy(sim_hbm.at[pl.ds(base, _ROWS_PER_W)], sim_tile)

    lane = lax.iota(jnp.int32, _LANES)
    mask8 = lane < TOPK

    def row_body(r, _):
        nodes = []
        for j in range(NUM_VARS // _LANES):
            kj = sim_tile[r, pl.ds(j * _LANES, _LANES)]
            vj = lane + (j * _LANES)
            nodes.append(plsc.sort_key_val(kj, vj, descending=True))
        while len(nodes) > 1:
            nxt = []
            for i in range(0, len(nodes), 2):
                ak, av = nodes[i]
                bk, bv = nodes[i + 1]
                mk = jnp.where(mask8, ak, lax.rev(bk, (0,)))
                mv = jnp.where(mask8, av, lax.rev(bv, (0,)))
                nxt.append(plsc.sort_key_val(mk, mv, descending=True))
            nodes = nxt
        kf, vf = nodes[0]

        m0 = jnp.max(kf)
        e = jnp.where(mask8, jnp.exp((kf - m0) * jnp.float32(1.0 / TEMP)),
                      jnp.float32(0.0))
        denom = jnp.broadcast_to(jnp.sum(e), (_LANES,))
        w = e / denom

        zero = jnp.zeros((_LANES,), jnp.float32)
        for j in range(NUM_VARS // _LANES):
            s_tile[r, pl.ds(j * _LANES, _LANES)] = zero
        plsc.store_scatter(s_tile, [jnp.full((_LANES,), r, jnp.int32), vf],
                           w, mask=mask8)
        return ()

    lax.fori_loop(0, _ROWS_PER_W, row_body, ())
    pltpu.sync_copy(s_tile, s_hbm.at[pl.ds(base, _ROWS_PER_W)])


_sc_routing = functools.partial(
    pl.kernel,
    out_type=jax.ShapeDtypeStruct((NUM_VARS, NUM_VARS), jnp.float32),
    mesh=plsc.VectorSubcoreMesh(core_axis_name="c", subcore_axis_name="s"),
    compiler_params=pltpu.CompilerParams(needs_layout_passes=False),
    scratch_types=[
        pltpu.VMEM((_ROWS_PER_W, NUM_VARS), jnp.float32),
        pltpu.VMEM((_ROWS_PER_W, NUM_VARS), jnp.float32),
    ],
)(_sc_routing_body)


def _combine_kernel(x_ref, s_ref, o_ref):
    o_ref[...] = lax.dot_general(
        x_ref[...], s_ref[...], (((1,), (1,)), ((), ())),
        preferred_element_type=jnp.float32)


@jax.jit
def kernel(x, var_embed, Wq, bq, Wk, bk):
    Bsz, L, N = x.shape
    ve = var_embed.reshape(N, HIDDEN)

    sim = pl.pallas_call(
        _sim_kernel,
        out_shape=jax.ShapeDtypeStruct((N, N), jnp.float32),
    )(ve, Wq, bq.reshape(1, HIDDEN), Wk, bk.reshape(1, HIDDEN))

    s = _sc_routing(sim)

    xs = x.reshape(Bsz * L, N)
    BL = 4096
    grid = (Bsz * L) // BL
    out = pl.pallas_call(
        _combine_kernel,
        grid=(grid,),
        in_specs=[
            pl.BlockSpec((BL, N), lambda i: (i, 0)),
            pl.BlockSpec((N, N), lambda i: (0, 0)),
        ],
        out_specs=pl.BlockSpec((BL, N), lambda i: (i, 0)),
        out_shape=jax.ShapeDtypeStruct((Bsz * L, N), jnp.float32),
    )(xs, s)
    return out.reshape(Bsz, L, N)
